# baseline (speedup 1.0000x reference)
import functools

import jax
import jax.numpy as jnp
from jax import lax
from jax.experimental import pallas as pl
from jax.experimental.pallas import tpu as pltpu
from jax.experimental.pallas import tpu_sc as plsc

BATCH = 16384
L = 20
LPAD = 128
EMBED = 64
HIDDEN = 128

NC = 2
NS = 16
NW = NC * NS

TASKS_PER_SRC_W = BATCH // NW
CHUNK = 32
NCHUNK = TASKS_PER_SRC_W // CHUNK
ROWS_PER_CHUNK = CHUNK * L
PAD_PER_CHUNK = CHUNK * LPAD
GATHER_SLICE = 128
NSLICE = ROWS_PER_CHUNK // GATHER_SLICE
NPOS = ROWS_PER_CHUNK // 16


def _pool_kernel(a_hbm, b_hbm, table_hbm, out_hbm,
                 pad_v, idx_v, pos_v, rows_v, out_v, sem):
    wid = lax.axis_index("s") * NC + lax.axis_index("c")

    for k in range(NPOS):
        i = lax.iota(jnp.int32, 16) + (16 * k)
        q = lax.shift_right_logical(i * 3277, 16)
        pos_v[pl.ds(16 * k, 16)] = q * (LPAD - L) + i

    for src_hbm, col0 in ((a_hbm, 0), (b_hbm, EMBED)):
        def chunk_body(c, _):
            task0 = wid * TASKS_PER_SRC_W + c * CHUNK
            pad_off = pl.multiple_of(task0 * LPAD, PAD_PER_CHUNK)
            pltpu.sync_copy(src_hbm.at[pl.ds(pad_off, PAD_PER_CHUNK)], pad_v)
            for k in range(NPOS):
                pos = pos_v[pl.ds(16 * k, 16)]
                idx_v[pl.ds(16 * k, 16)] = plsc.load_gather(pad_v, [pos])
            copies = [
                pltpu.async_copy(
                    table_hbm.at[idx_v.at[pl.ds(j * GATHER_SLICE, GATHER_SLICE)]],
                    rows_v.at[pl.ds(j * GATHER_SLICE, GATHER_SLICE)],
                    sem,
                )
                for j in range(NSLICE)
            ]
            for cp in copies:
                cp.wait()

            def task_body(t, _):
                for g in range(EMBED // 16):
                    acc = rows_v[t * L, pl.ds(g * 16, 16)]
                    for r in range(1, L):
                        acc = acc + rows_v[t * L + r, pl.ds(g * 16, 16)]
                    out_v[t, pl.ds(g * 16, 16)] = acc * (1.0 / L)
                return 0

            lax.fori_loop(0, CHUNK, task_body, 0)
            pltpu.sync_copy(
                out_v, out_hbm.at[pl.ds(task0, CHUNK), pl.ds(col0, EMBED)])
            return 0

        lax.fori_loop(0, NCHUNK, chunk_body, 0)


@functools.partial(
    pl.kernel,
    mesh=plsc.VectorSubcoreMesh(core_axis_name="c", subcore_axis_name="s"),
    out_type=jax.ShapeDtypeStruct((BATCH, 2 * EMBED), jnp.float32),
    compiler_params=pltpu.CompilerParams(
        use_tc_tiling_on_sc=False, needs_layout_passes=False),
    scratch_types=[
        pltpu.VMEM((PAD_PER_CHUNK,), jnp.int32),
        pltpu.VMEM((ROWS_PER_CHUNK,), jnp.int32),
        pltpu.VMEM((ROWS_PER_CHUNK,), jnp.int32),
        pltpu.VMEM((ROWS_PER_CHUNK, EMBED), jnp.float32),
        pltpu.VMEM((CHUNK, EMBED), jnp.float32),
        pltpu.SemaphoreType.DMA,
    ],
)
def _pool(a_hbm, b_hbm, table_hbm, out_hbm,
          pad_v, idx_v, pos_v, rows_v, out_v, sem):
    _pool_kernel(a_hbm, b_hbm, table_hbm, out_hbm,
                 pad_v, idx_v, pos_v, rows_v, out_v, sem)


MLP_TILE = 512


def _mlp_body(x_ref, w1_ref, b1_ref, w2_ref, b2_ref, w3_ref, b3_ref, out_ref):
    x = x_ref[...]
    h = jnp.dot(x, w1_ref[...], preferred_element_type=jnp.float32) + b1_ref[...]
    h = jnp.maximum(h, 0.0)
    h = jnp.dot(h, w2_ref[...], preferred_element_type=jnp.float32) + b2_ref[...]
    h = jnp.maximum(h, 0.0)
    logit = jnp.sum(h * w3_ref[...], axis=1) + b3_ref[0, 0]
    out_ref[0, :] = jax.nn.sigmoid(logit)


def _mlp(x, w1t, b1, w2t, b2, w3, b3):
    grid = (BATCH // MLP_TILE,)
    full = lambda i: (0, 0)
    out = pl.pallas_call(
        _mlp_body,
        grid=grid,
        in_specs=[
            pl.BlockSpec((MLP_TILE, 2 * EMBED), lambda i: (i, 0)),
            pl.BlockSpec((2 * EMBED, HIDDEN), full),
            pl.BlockSpec((1, HIDDEN), full),
            pl.BlockSpec((HIDDEN, HIDDEN), full),
            pl.BlockSpec((1, HIDDEN), full),
            pl.BlockSpec((1, HIDDEN), full),
            pl.BlockSpec((1, 1), full),
        ],
        out_specs=pl.BlockSpec((1, MLP_TILE), lambda i: (0, i)),
        out_shape=jax.ShapeDtypeStruct((1, BATCH), jnp.float32),
    )(x, w1t, b1.reshape(1, HIDDEN), w2t, b2.reshape(1, HIDDEN),
      w3.reshape(1, HIDDEN), b3.reshape(1, 1))
    return out[0]


def _pad_flat(idx):
    idx = idx.astype(jnp.int32)
    return jnp.pad(idx, ((0, 0), (0, LPAD - L))).reshape(-1)


def kernel(a_indices_list, b_indices_list, table, W1, b1, W2, b2, W3, b3):
    pooled = _pool(_pad_flat(a_indices_list), _pad_flat(b_indices_list), table)
    return _mlp(pooled, W1.T, b1, W2.T, b2, W3, b3)


# speedup vs baseline: 1.0805x; 1.0805x over previous
import functools

import jax
import jax.numpy as jnp
from jax import lax
from jax.experimental import pallas as pl
from jax.experimental.pallas import tpu as pltpu
from jax.experimental.pallas import tpu_sc as plsc

BATCH = 16384
L = 20
LPAD = 128
EMBED = 64
HIDDEN = 128

NC = 2
NS = 16
NW = NC * NS

TASKS_PER_SRC_W = BATCH // NW
CHUNK = 32
NCHUNK = TASKS_PER_SRC_W // CHUNK
ROWS_PER_CHUNK = CHUNK * L
PAD_PER_CHUNK = CHUNK * LPAD
GATHER_SLICE = 128
NSLICE = ROWS_PER_CHUNK // GATHER_SLICE
NPOS = ROWS_PER_CHUNK // 16


def _pool_kernel(a_hbm, b_hbm, table_hbm, out_hbm,
                 pad_v, idx_v, pos_v, rows_v, out_v, sem):
    wid = lax.axis_index("s") * NC + lax.axis_index("c")

    for k in range(NPOS):
        i = lax.iota(jnp.int32, 16) + (16 * k)
        q = lax.shift_right_logical(i * 3277, 16)
        pos_v[pl.ds(16 * k, 16)] = q * (LPAD - L) + i

    for src_hbm, col0 in ((a_hbm, 0), (b_hbm, EMBED)):
        def chunk_body(c, _):
            task0 = wid * TASKS_PER_SRC_W + c * CHUNK
            pad_off = pl.multiple_of(task0 * LPAD, PAD_PER_CHUNK)
            pltpu.sync_copy(src_hbm.at[pl.ds(pad_off, PAD_PER_CHUNK)], pad_v)
            for k in range(NPOS):
                pos = pos_v[pl.ds(16 * k, 16)]
                idx_v[pl.ds(16 * k, 16)] = plsc.load_gather(pad_v, [pos]) * 2
            copies = [
                pltpu.async_copy(
                    table_hbm.at[idx_v.at[pl.ds(j * GATHER_SLICE, GATHER_SLICE)]],
                    rows_v.at[pl.ds(j * GATHER_SLICE, GATHER_SLICE)],
                    sem,
                )
                for j in range(NSLICE)
            ]
            for cp in copies:
                cp.wait()

            def task_body(t, _):
                for g in range(EMBED // 16):
                    acc = rows_v[t * L, pl.ds(g * 16, 16)]
                    for r in range(1, L):
                        acc = acc + rows_v[t * L + r, pl.ds(g * 16, 16)]
                    out_v[t, pl.ds(g * 16, 16)] = acc * (1.0 / L)
                return 0

            lax.fori_loop(0, CHUNK, task_body, 0)
            pltpu.sync_copy(
                out_v, out_hbm.at[pl.ds(task0, CHUNK), pl.ds(col0, EMBED)])
            return 0

        lax.fori_loop(0, NCHUNK, chunk_body, 0)


@functools.partial(
    pl.kernel,
    mesh=plsc.VectorSubcoreMesh(core_axis_name="c", subcore_axis_name="s"),
    out_type=jax.ShapeDtypeStruct((BATCH, 2 * EMBED), jnp.float32),
    compiler_params=pltpu.CompilerParams(
        use_tc_tiling_on_sc=False, needs_layout_passes=False),
    scratch_types=[
        pltpu.VMEM((PAD_PER_CHUNK,), jnp.int32),
        pltpu.VMEM((ROWS_PER_CHUNK,), jnp.int32),
        pltpu.VMEM((ROWS_PER_CHUNK,), jnp.int32),
        pltpu.VMEM((ROWS_PER_CHUNK, EMBED), jnp.float32),
        pltpu.VMEM((CHUNK, EMBED), jnp.float32),
        pltpu.SemaphoreType.DMA,
    ],
)
def _pool(a_hbm, b_hbm, table_hbm, out_hbm,
          pad_v, idx_v, pos_v, rows_v, out_v, sem):
    _pool_kernel(a_hbm, b_hbm, table_hbm, out_hbm,
                 pad_v, idx_v, pos_v, rows_v, out_v, sem)


MLP_TILE = 512


def _mlp_body(x_ref, w1_ref, b1_ref, w2_ref, b2_ref, w3_ref, b3_ref, out_ref):
    x = x_ref[...]
    h = jnp.dot(x, w1_ref[...], preferred_element_type=jnp.float32) + b1_ref[...]
    h = jnp.maximum(h, 0.0)
    h = jnp.dot(h, w2_ref[...], preferred_element_type=jnp.float32) + b2_ref[...]
    h = jnp.maximum(h, 0.0)
    logit = jnp.sum(h * w3_ref[...], axis=1) + b3_ref[0, 0]
    out_ref[0, :] = jax.nn.sigmoid(logit)


def _mlp(x, w1t, b1, w2t, b2, w3, b3):
    grid = (BATCH // MLP_TILE,)
    full = lambda i: (0, 0)
    out = pl.pallas_call(
        _mlp_body,
        grid=grid,
        in_specs=[
            pl.BlockSpec((MLP_TILE, 2 * EMBED), lambda i: (i, 0)),
            pl.BlockSpec((2 * EMBED, HIDDEN), full),
            pl.BlockSpec((1, HIDDEN), full),
            pl.BlockSpec((HIDDEN, HIDDEN), full),
            pl.BlockSpec((1, HIDDEN), full),
            pl.BlockSpec((1, HIDDEN), full),
            pl.BlockSpec((1, 1), full),
        ],
        out_specs=pl.BlockSpec((1, MLP_TILE), lambda i: (0, i)),
        out_shape=jax.ShapeDtypeStruct((1, BATCH), jnp.float32),
    )(x, w1t, b1.reshape(1, HIDDEN), w2t, b2.reshape(1, HIDDEN),
      w3.reshape(1, HIDDEN), b3.reshape(1, 1))
    return out[0]


def _pad_flat(idx):
    idx = idx.astype(jnp.int32)
    return jnp.pad(idx, ((0, 0), (0, LPAD - L))).reshape(-1)


def kernel(a_indices_list, b_indices_list, table, W1, b1, W2, b2, W3, b3):
    tpad = jnp.pad(table, ((0, 0), (0, EMBED))).reshape(2 * table.shape[0], EMBED)
    pooled = _pool(_pad_flat(a_indices_list), _pad_flat(b_indices_list), tpad)
    return _mlp(pooled, W1.T, b1, W2.T, b2, W3, b3)


# speedup vs baseline: 1.4276x; 1.3212x over previous
import functools

import jax
import jax.numpy as jnp
from jax import lax
from jax.experimental import pallas as pl
from jax.experimental.pallas import tpu as pltpu
from jax.experimental.pallas import tpu_sc as plsc

BATCH = 16384
L = 20
LPAD = 128
EMBED = 64
HIDDEN = 128

NC = 2
NS = 16
NW = NC * NS

TASKS_PER_SRC_W = BATCH // NW
CHUNK = 32
NCHUNK = TASKS_PER_SRC_W // CHUNK
ROWS_PER_CHUNK = CHUNK * L
PAD_PER_CHUNK = CHUNK * LPAD
GATHER_SLICE = 128
NSLICE = ROWS_PER_CHUNK // GATHER_SLICE
NPOS = ROWS_PER_CHUNK // 16


def _pool_kernel(a_hbm, b_hbm, table_hbm, out_hbm,
                 pad_v, idx_v, pos_v, rows_v, out_v, sem):
    wid = lax.axis_index("s") * NC + lax.axis_index("c")

    for k in range(NPOS):
        i = lax.iota(jnp.int32, 16) + (16 * k)
        q = lax.shift_right_logical(i * 3277, 16)
        pos_v[pl.ds(16 * k, 16)] = q * (LPAD - L) + i

    for src_hbm, col0 in ((a_hbm, 0), (b_hbm, EMBED)):
        def chunk_body(c, _):
            task0 = wid * TASKS_PER_SRC_W + c * CHUNK
            pad_off = pl.multiple_of(task0 * LPAD, PAD_PER_CHUNK)
            pltpu.sync_copy(src_hbm.at[pl.ds(pad_off, PAD_PER_CHUNK)], pad_v)
            for k in range(NPOS):
                pos = pos_v[pl.ds(16 * k, 16)]
                v = plsc.load_gather(pad_v, [pos])
                idx_v[pl.ds(16 * k, 16)] = (
                    (v & ~jnp.int32(TP_BLK - 1))
                    | lax.shift_left(v & jnp.int32(TP_HALF - 1), 1)
                    | (lax.shift_right_logical(v, 11) & jnp.int32(1))
                )
            copies = [
                pltpu.async_copy(
                    table_hbm.at[idx_v.at[pl.ds(j * GATHER_SLICE, GATHER_SLICE)]],
                    rows_v.at[pl.ds(j * GATHER_SLICE, GATHER_SLICE)],
                    sem,
                )
                for j in range(NSLICE)
            ]
            for cp in copies:
                cp.wait()

            def task_body(t, _):
                for g in range(EMBED // 16):
                    acc = rows_v[t * L, pl.ds(g * 16, 16)]
                    for r in range(1, L):
                        acc = acc + rows_v[t * L + r, pl.ds(g * 16, 16)]
                    out_v[t, pl.ds(g * 16, 16)] = acc * (1.0 / L)
                return 0

            lax.fori_loop(0, CHUNK, task_body, 0)
            pltpu.sync_copy(
                out_v, out_hbm.at[pl.ds(task0, CHUNK), pl.ds(col0, EMBED)])
            return 0

        lax.fori_loop(0, NCHUNK, chunk_body, 0)


@functools.partial(
    pl.kernel,
    mesh=plsc.VectorSubcoreMesh(core_axis_name="c", subcore_axis_name="s"),
    out_type=jax.ShapeDtypeStruct((BATCH, 2 * EMBED), jnp.float32),
    compiler_params=pltpu.CompilerParams(
        use_tc_tiling_on_sc=False, needs_layout_passes=False),
    scratch_types=[
        pltpu.VMEM((PAD_PER_CHUNK,), jnp.int32),
        pltpu.VMEM((ROWS_PER_CHUNK,), jnp.int32),
        pltpu.VMEM((ROWS_PER_CHUNK,), jnp.int32),
        pltpu.VMEM((ROWS_PER_CHUNK, EMBED), jnp.float32),
        pltpu.VMEM((CHUNK, EMBED), jnp.float32),
        pltpu.SemaphoreType.DMA,
    ],
)
def _pool(a_hbm, b_hbm, table_hbm, out_hbm,
          pad_v, idx_v, pos_v, rows_v, out_v, sem):
    _pool_kernel(a_hbm, b_hbm, table_hbm, out_hbm,
                 pad_v, idx_v, pos_v, rows_v, out_v, sem)


MLP_TILE = 512


def _mlp_body(x_ref, w1_ref, b1_ref, w2_ref, b2_ref, w3_ref, b3_ref, out_ref):
    x = x_ref[...]
    h = jnp.dot(x, w1_ref[...], preferred_element_type=jnp.float32) + b1_ref[...]
    h = jnp.maximum(h, 0.0)
    h = jnp.dot(h, w2_ref[...], preferred_element_type=jnp.float32) + b2_ref[...]
    h = jnp.maximum(h, 0.0)
    logit = jnp.sum(h * w3_ref[...], axis=1) + b3_ref[0, 0]
    out_ref[0, :] = jax.nn.sigmoid(logit)


def _mlp(x, w1t, b1, w2t, b2, w3, b3):
    grid = (BATCH // MLP_TILE,)
    full = lambda i: (0, 0)
    out = pl.pallas_call(
        _mlp_body,
        grid=grid,
        in_specs=[
            pl.BlockSpec((MLP_TILE, 2 * EMBED), lambda i: (i, 0)),
            pl.BlockSpec((2 * EMBED, HIDDEN), full),
            pl.BlockSpec((1, HIDDEN), full),
            pl.BlockSpec((HIDDEN, HIDDEN), full),
            pl.BlockSpec((1, HIDDEN), full),
            pl.BlockSpec((1, HIDDEN), full),
            pl.BlockSpec((1, 1), full),
        ],
        out_specs=pl.BlockSpec((1, MLP_TILE), lambda i: (0, i)),
        out_shape=jax.ShapeDtypeStruct((1, BATCH), jnp.float32),
    )(x, w1t, b1.reshape(1, HIDDEN), w2t, b2.reshape(1, HIDDEN),
      w3.reshape(1, HIDDEN), b3.reshape(1, 1))
    return out[0]


TP_BLK = 4096
TP_HALF = TP_BLK // 2


def _tpose_body(in_ref, out_ref):
    x = in_ref[...]
    out_ref[...] = jnp.concatenate(
        [x[:, :TP_HALF].T, x[:, TP_HALF:].T], axis=1)


def _tpose(tableT):
    n = tableT.shape[1]
    grid = (n + TP_BLK - 1) // TP_BLK
    return pl.pallas_call(
        _tpose_body,
        grid=(grid,),
        in_specs=[pl.BlockSpec((EMBED, TP_BLK), lambda i: (0, i))],
        out_specs=pl.BlockSpec((TP_HALF, 2 * EMBED), lambda i: (i, 0)),
        out_shape=jax.ShapeDtypeStruct((grid * TP_HALF, 2 * EMBED), jnp.float32),
    )(tableT)


def _pad_flat(idx):
    idx = idx.astype(jnp.int32)
    return jnp.pad(idx, ((0, 0), (0, LPAD - L))).reshape(-1)


def kernel(a_indices_list, b_indices_list, table, W1, b1, W2, b2, W3, b3):
    tlin = _tpose(table.T)
    tlin = tlin.reshape(tlin.shape[0] * 2, EMBED)
    pooled = _pool(_pad_flat(a_indices_list), _pad_flat(b_indices_list), tlin)
    return _mlp(pooled, W1.T, b1, W2.T, b2, W3, b3)
